# baseline (speedup 1.0000x reference)
import functools

import jax
import jax.numpy as jnp
from jax import lax
from jax.experimental import pallas as pl
from jax.experimental.pallas import tpu as pltpu
from jax.experimental.pallas import tpu_sc as plsc

N, Q, P, S, B, D = 32768, 256, 10, 8, 16, 64
NC, NS, L = 2, 16, 16
NW = NC * NS
RPW = N // NW
CHUNK = 128
NCHUNK = RPW // CHUNK

_F32 = jnp.float32

_GDN = lax.GatherDimensionNumbers(
    offset_dims=(), collapsed_slice_dims=(0,), start_index_map=(0,))


def _splat_lane(vec, lane):
    idx = jnp.full((L, 1), lane, jnp.int32)
    return lax.gather(vec, idx, _GDN, slice_sizes=(1,),
                      mode=lax.GatherScatterMode.PROMISE_IN_BOUNDS)


def _hsum_splat(vec):
    x = vec
    for stride in (1, 2, 4, 8):
        idx = (jnp.arange(L, dtype=jnp.int32) ^ stride).reshape(L, 1)
        x = x + lax.gather(x, idx, _GDN, slice_sizes=(1,),
                           mode=lax.GatherScatterMode.PROMISE_IN_BOUNDS)
    return x


def _sort16(v):
    return jnp.sort(v)


def _sort64(d0, d1, d2, d3):
    s0 = _sort16(d0)
    s1d = -_sort16(-d1)
    la, ha = jnp.minimum(s0, s1d), jnp.maximum(s0, s1d)
    a0, a1 = _sort16(la), _sort16(ha)
    s2 = _sort16(d2)
    s3d = -_sort16(-d3)
    lb, hb = jnp.minimum(s2, s3d), jnp.maximum(s2, s3d)
    b0, b1 = -_sort16(-hb), -_sort16(-lb)
    l0, l1 = jnp.minimum(a0, b0), jnp.minimum(a1, b1)
    h0, h1 = jnp.maximum(a0, b0), jnp.maximum(a1, b1)
    o0 = _sort16(jnp.minimum(l0, l1))
    o1 = _sort16(jnp.maximum(l0, l1))
    o2 = _sort16(jnp.minimum(h0, h1))
    o3 = _sort16(jnp.maximum(h0, h1))
    return o0, o1, o2, o3


_LANE_IOTA = tuple(range(L))


def _sc_body(x_hbm, xd_hbm, topo_hbm, seg_hbm, out_hbm,
             x_v0, x_v1, xd_v0, xd_v1, sid_v0, sid_v1,
             topo_v, acc_v, dsum_v, u_v, sem0, sem1):
    cid = lax.axis_index("c")
    sid = lax.axis_index("s")
    wid = cid * NS + sid

    pltpu.sync_copy(topo_hbm, topo_v)
    t = [topo_v[pl.ds(k * L, L)] for k in range(4)]
    e = [jnp.exp(tk) for tk in t]
    sden = _hsum_splat(e[0] + e[1] + e[2] + e[3])
    v = [ek / sden for ek in e]
    r0, r1, r2, r3 = _sort64(v[0], v[1], v[2], v[3])

    def zrow(r, carry):
        for cc in range(Q // L):
            acc_v[r, pl.ds(cc * L, L)] = jnp.zeros((L,), _F32)
        return carry
    lax.fori_loop(0, B, zrow, 0)

    base = wid * RPW
    col_iota = jnp.arange(L, dtype=jnp.int32)

    bufs = ((x_v0, xd_v0, sid_v0, sem0), (x_v1, xd_v1, sid_v1, sem1))

    def _start(k):
        xb, db, sb, sem = bufs[k % 2]
        row0 = base + k * CHUNK
        return (pltpu.async_copy(x_hbm.at[pl.ds(row0, CHUNK)], xb, sem),
                pltpu.async_copy(xd_hbm.at[pl.ds(row0, CHUNK)], db, sem),
                pltpu.async_copy(seg_hbm.at[pl.ds(row0, CHUNK)], sb, sem))

    handles = _start(0)
    for k in range(NCHUNK):
        nxt = _start(k + 1) if k + 1 < NCHUNK else None
        for h in handles:
            h.wait()
        x_v, xd_v, sid_v, _ = bufs[k % 2]

        @plsc.parallel_loop(0, CHUNK, 1, unroll=4)
        def sort_body(r):
            d0 = xd_v[r, pl.ds(0 * L, L)]
            d1 = xd_v[r, pl.ds(1 * L, L)]
            d2 = xd_v[r, pl.ds(2 * L, L)]
            d3 = xd_v[r, pl.ds(3 * L, L)]
            s0, s1, s2, s3 = _sort64(d0, d1, d2, d3)
            dsum_v[r, pl.ds(0, L)] = (
                jnp.abs(s0 - r0) + jnp.abs(s1 - r1)
                + jnp.abs(s2 - r2) + jnp.abs(s3 - r3))

        def wsum_body(g, carry):
            rows = jnp.broadcast_to(g * L, (L,)) + col_iota
            tot = plsc.load_gather(dsum_v, [rows, jnp.zeros((L,), jnp.int32)])
            for p in range(1, L):
                tot = tot + plsc.load_gather(
                    dsum_v, [rows, jnp.full((L,), p, jnp.int32)])
            u_v[pl.ds(g * L, L)] = jnp.exp(tot * (-1.0 / D))
            return carry

        lax.fori_loop(0, CHUNK // L, wsum_body, 0)

        @plsc.parallel_loop(0, CHUNK, 1, unroll=4)
        def scatter_body(r):
            rr = jnp.broadcast_to(r, (L,))
            bag = plsc.load_gather(sid_v, [rr])
            ub = plsc.load_gather(u_v, [rr])
            for cc in range(Q // L):
                plsc.addupdate_scatter(
                    acc_v, [bag, col_iota + (cc * L)],
                    x_v[r, pl.ds(cc * L, L)] * ub)

        handles = nxt

    pltpu.sync_copy(acc_v, out_hbm.at[wid])


def _sc_segment_weighted_sum(x, x_dists, topology, segment_ids):
    mesh = plsc.VectorSubcoreMesh(core_axis_name="c", subcore_axis_name="s")
    kern = functools.partial(
        pl.kernel,
        out_type=jax.ShapeDtypeStruct((NW, B, Q), _F32),
        mesh=mesh,
        compiler_params=pltpu.CompilerParams(needs_layout_passes=False),
        scratch_types=[
            pltpu.VMEM((CHUNK, Q), _F32),
            pltpu.VMEM((CHUNK, Q), _F32),
            pltpu.VMEM((CHUNK, D), _F32),
            pltpu.VMEM((CHUNK, D), _F32),
            pltpu.VMEM((CHUNK,), jnp.int32),
            pltpu.VMEM((CHUNK,), jnp.int32),
            pltpu.VMEM((D,), _F32),
            pltpu.VMEM((B, Q), _F32),
            pltpu.VMEM((CHUNK, L), _F32),
            pltpu.VMEM((CHUNK,), _F32),
            pltpu.SemaphoreType.DMA,
            pltpu.SemaphoreType.DMA,
        ],
    )(_sc_body)
    return kern(x, x_dists, topology, segment_ids)


def _tail_body(part_ref, bz_ref, out_ref):
    z16 = jnp.sum(part_ref[...], axis=0)
    bz = bz_ref[...]
    y = jnp.dot(z16, bz, preferred_element_type=_F32)
    b2 = jnp.sqrt(jnp.mean(bz * bz, axis=0, keepdims=True))
    m = jnp.mean(y, axis=0, keepdims=True)
    c = y - m
    var = jnp.sum(c * c, axis=0, keepdims=True) * (1.0 / (B - 1))
    out_ref[...] = b2 * c * lax.rsqrt(var)


def kernel(x, x_dists, topology, beta_z, segment_ids):
    partials = _sc_segment_weighted_sum(
        x, x_dists, topology, segment_ids.astype(jnp.int32))
    bz2 = beta_z.reshape(Q, P * S)
    y = pl.pallas_call(
        _tail_body,
        out_shape=jax.ShapeDtypeStruct((B, P * S), _F32),
    )(partials, bz2)
    return y.reshape(B, P, S)


# speedup vs baseline: 1.1320x; 1.1320x over previous
import functools

import jax
import jax.numpy as jnp
from jax import lax
from jax.experimental import pallas as pl
from jax.experimental.pallas import tpu as pltpu
from jax.experimental.pallas import tpu_sc as plsc

N, Q, P, S, B, D = 32768, 256, 10, 8, 16, 64
NC, NS, L = 2, 16, 16
NW = NC * NS
RPW = N // NW
CHUNK = 128
NCHUNK = RPW // CHUNK

_F32 = jnp.float32

_GDN = lax.GatherDimensionNumbers(
    offset_dims=(), collapsed_slice_dims=(0,), start_index_map=(0,))


def _splat_lane(vec, lane):
    idx = jnp.full((L, 1), lane, jnp.int32)
    return lax.gather(vec, idx, _GDN, slice_sizes=(1,),
                      mode=lax.GatherScatterMode.PROMISE_IN_BOUNDS)


def _hsum_splat(vec):
    x = vec
    for stride in (1, 2, 4, 8):
        idx = (jnp.arange(L, dtype=jnp.int32) ^ stride).reshape(L, 1)
        x = x + lax.gather(x, idx, _GDN, slice_sizes=(1,),
                           mode=lax.GatherScatterMode.PROMISE_IN_BOUNDS)
    return x


def _sort16(v):
    return jnp.sort(v)


def _sort64(d0, d1, d2, d3):
    s0 = _sort16(d0)
    s1d = -_sort16(-d1)
    la, ha = jnp.minimum(s0, s1d), jnp.maximum(s0, s1d)
    a0, a1 = _sort16(la), _sort16(ha)
    s2 = _sort16(d2)
    s3d = -_sort16(-d3)
    lb, hb = jnp.minimum(s2, s3d), jnp.maximum(s2, s3d)
    b0, b1 = -_sort16(-hb), -_sort16(-lb)
    l0, l1 = jnp.minimum(a0, b0), jnp.minimum(a1, b1)
    h0, h1 = jnp.maximum(a0, b0), jnp.maximum(a1, b1)
    o0 = _sort16(jnp.minimum(l0, l1))
    o1 = _sort16(jnp.maximum(l0, l1))
    o2 = _sort16(jnp.minimum(h0, h1))
    o3 = _sort16(jnp.maximum(h0, h1))
    return o0, o1, o2, o3


_LANE_IOTA = tuple(range(L))


def _sc_body(x_hbm, xd_hbm, topo_hbm, seg_hbm, out_hbm,
             x_v0, x_v1, xd_v0, xd_v1, sid_v0, sid_v1,
             topo_v, acc_v, sem0, sem1):
    cid = lax.axis_index("c")
    sid = lax.axis_index("s")
    wid = cid * NS + sid

    pltpu.sync_copy(topo_hbm, topo_v)
    t = [topo_v[pl.ds(k * L, L)] for k in range(4)]
    e = [jnp.exp(tk) for tk in t]
    sden = _hsum_splat(e[0] + e[1] + e[2] + e[3])
    v = [ek / sden for ek in e]
    r0, r1, r2, r3 = _sort64(v[0], v[1], v[2], v[3])

    def zrow(r, carry):
        for cc in range(Q // L):
            acc_v[r, pl.ds(cc * L, L)] = jnp.zeros((L,), _F32)
        return carry
    lax.fori_loop(0, B, zrow, 0)

    base = wid * RPW
    col_iota = jnp.arange(L, dtype=jnp.int32)

    bufs = ((x_v0, xd_v0, sid_v0, sem0), (x_v1, xd_v1, sid_v1, sem1))

    def _start(k):
        xb, db, sb, sem = bufs[k % 2]
        row0 = base + k * CHUNK
        return (pltpu.async_copy(x_hbm.at[pl.ds(row0, CHUNK)], xb, sem),
                pltpu.async_copy(xd_hbm.at[pl.ds(row0, CHUNK)], db, sem),
                pltpu.async_copy(seg_hbm.at[pl.ds(row0, CHUNK)], sb, sem))

    handles = _start(0)
    for k in range(NCHUNK):
        nxt = _start(k + 1) if k + 1 < NCHUNK else None
        for h in handles:
            h.wait()
        x_v, xd_v, sid_v, _ = bufs[k % 2]

        @plsc.parallel_loop(0, CHUNK, 1, unroll=3)
        def row_body(r):
            d0 = xd_v[r, pl.ds(0 * L, L)]
            d1 = xd_v[r, pl.ds(1 * L, L)]
            d2 = xd_v[r, pl.ds(2 * L, L)]
            d3 = xd_v[r, pl.ds(3 * L, L)]
            s0, s1, s2, s3 = _sort64(d0, d1, d2, d3)
            dsum = (jnp.abs(s0 - r0) + jnp.abs(s1 - r1)
                    + jnp.abs(s2 - r2) + jnp.abs(s3 - r3))
            u = jnp.exp(_hsum_splat(dsum) * (-1.0 / D))
            rr = jnp.broadcast_to(r, (L,))
            bag = plsc.load_gather(sid_v, [rr])
            for cc in range(Q // L):
                plsc.addupdate_scatter(
                    acc_v, [bag, col_iota + (cc * L)],
                    x_v[r, pl.ds(cc * L, L)] * u)

        handles = nxt

    pltpu.sync_copy(acc_v, out_hbm.at[wid])


def _sc_segment_weighted_sum(x, x_dists, topology, segment_ids):
    mesh = plsc.VectorSubcoreMesh(core_axis_name="c", subcore_axis_name="s")
    kern = functools.partial(
        pl.kernel,
        out_type=jax.ShapeDtypeStruct((NW, B, Q), _F32),
        mesh=mesh,
        compiler_params=pltpu.CompilerParams(needs_layout_passes=False),
        scratch_types=[
            pltpu.VMEM((CHUNK, Q), _F32),
            pltpu.VMEM((CHUNK, Q), _F32),
            pltpu.VMEM((CHUNK, D), _F32),
            pltpu.VMEM((CHUNK, D), _F32),
            pltpu.VMEM((CHUNK,), jnp.int32),
            pltpu.VMEM((CHUNK,), jnp.int32),
            pltpu.VMEM((D,), _F32),
            pltpu.VMEM((B, Q), _F32),
            pltpu.SemaphoreType.DMA,
            pltpu.SemaphoreType.DMA,
        ],
    )(_sc_body)
    return kern(x, x_dists, topology, segment_ids)


def _tail_body(part_ref, bz_ref, out_ref):
    z16 = jnp.sum(part_ref[...], axis=0)
    bz = bz_ref[...]
    y = jnp.dot(z16, bz, preferred_element_type=_F32)
    b2 = jnp.sqrt(jnp.mean(bz * bz, axis=0, keepdims=True))
    m = jnp.mean(y, axis=0, keepdims=True)
    c = y - m
    var = jnp.sum(c * c, axis=0, keepdims=True) * (1.0 / (B - 1))
    out_ref[...] = b2 * c * lax.rsqrt(var)


def kernel(x, x_dists, topology, beta_z, segment_ids):
    partials = _sc_segment_weighted_sum(
        x, x_dists, topology, segment_ids.astype(jnp.int32))
    bz2 = beta_z.reshape(Q, P * S)
    y = pl.pallas_call(
        _tail_body,
        out_shape=jax.ShapeDtypeStruct((B, P * S), _F32),
    )(partials, bz2)
    return y.reshape(B, P, S)


# speedup vs baseline: 1.2324x; 1.0887x over previous
import functools

import jax
import jax.numpy as jnp
from jax import lax
from jax.experimental import pallas as pl
from jax.experimental.pallas import tpu as pltpu
from jax.experimental.pallas import tpu_sc as plsc

N, Q, P, S, B, D = 32768, 256, 10, 8, 16, 64
NC, NS, L = 2, 16, 16
NW = NC * NS
RPW = N // NW
CHUNK = 128
NCHUNK = RPW // CHUNK

_F32 = jnp.float32

_GDN = lax.GatherDimensionNumbers(
    offset_dims=(), collapsed_slice_dims=(0,), start_index_map=(0,))


def _splat_lane(vec, lane):
    idx = jnp.full((L, 1), lane, jnp.int32)
    return lax.gather(vec, idx, _GDN, slice_sizes=(1,),
                      mode=lax.GatherScatterMode.PROMISE_IN_BOUNDS)


def _hsum_splat(vec):
    x = vec
    for stride in (1, 2, 4, 8):
        idx = (jnp.arange(L, dtype=jnp.int32) ^ stride).reshape(L, 1)
        x = x + lax.gather(x, idx, _GDN, slice_sizes=(1,),
                           mode=lax.GatherScatterMode.PROMISE_IN_BOUNDS)
    return x


def _sort16(v):
    return jnp.sort(v)


def _sort16d(v):
    return plsc.sort_key_val(v, v, descending=True)[0]


def _sort64(d0, d1, d2, d3):
    s0 = _sort16(d0)
    s1d = _sort16d(d1)
    la, ha = jnp.minimum(s0, s1d), jnp.maximum(s0, s1d)
    a0, a1 = _sort16(la), _sort16(ha)
    s2 = _sort16(d2)
    s3d = _sort16d(d3)
    lb, hb = jnp.minimum(s2, s3d), jnp.maximum(s2, s3d)
    b0, b1 = _sort16d(hb), _sort16d(lb)
    l0, l1 = jnp.minimum(a0, b0), jnp.minimum(a1, b1)
    h0, h1 = jnp.maximum(a0, b0), jnp.maximum(a1, b1)
    o0 = _sort16(jnp.minimum(l0, l1))
    o1 = _sort16(jnp.maximum(l0, l1))
    o2 = _sort16(jnp.minimum(h0, h1))
    o3 = _sort16(jnp.maximum(h0, h1))
    return o0, o1, o2, o3


_LANE_IOTA = tuple(range(L))


def _sc_body(x_hbm, xd_hbm, topo_hbm, seg_hbm, out_hbm,
             x_v0, x_v1, xd_v0, xd_v1, sid_v0, sid_v1,
             topo_v, acc_v, sem0, sem1):
    cid = lax.axis_index("c")
    sid = lax.axis_index("s")
    wid = cid * NS + sid

    pltpu.sync_copy(topo_hbm, topo_v)
    t = [topo_v[pl.ds(k * L, L)] for k in range(4)]
    e = [jnp.exp(tk) for tk in t]
    sden = _hsum_splat(e[0] + e[1] + e[2] + e[3])
    v = [ek / sden for ek in e]
    r0, r1, r2, r3 = _sort64(v[0], v[1], v[2], v[3])

    def zrow(r, carry):
        for cc in range(Q // L):
            acc_v[r, pl.ds(cc * L, L)] = jnp.zeros((L,), _F32)
        return carry
    lax.fori_loop(0, B, zrow, 0)

    base = wid * RPW
    col_iota = jnp.arange(L, dtype=jnp.int32)

    bufs = ((x_v0, xd_v0, sid_v0, sem0), (x_v1, xd_v1, sid_v1, sem1))

    def _start(k):
        xb, db, sb, sem = bufs[k % 2]
        row0 = base + k * CHUNK
        return (pltpu.async_copy(x_hbm.at[pl.ds(row0, CHUNK)], xb, sem),
                pltpu.async_copy(xd_hbm.at[pl.ds(row0, CHUNK)], db, sem),
                pltpu.async_copy(seg_hbm.at[pl.ds(row0, CHUNK)], sb, sem))

    handles = _start(0)
    for k in range(NCHUNK):
        nxt = _start(k + 1) if k + 1 < NCHUNK else None
        for h in handles:
            h.wait()
        x_v, xd_v, sid_v, _ = bufs[k % 2]

        @plsc.parallel_loop(0, CHUNK, 1, unroll=2)
        def row_body(r):
            d0 = xd_v[r, pl.ds(0 * L, L)]
            d1 = xd_v[r, pl.ds(1 * L, L)]
            d2 = xd_v[r, pl.ds(2 * L, L)]
            d3 = xd_v[r, pl.ds(3 * L, L)]
            s0, s1, s2, s3 = _sort64(d0, d1, d2, d3)
            dsum = (jnp.abs(s0 - r0) + jnp.abs(s1 - r1)
                    + jnp.abs(s2 - r2) + jnp.abs(s3 - r3))
            u = jnp.exp(_hsum_splat(dsum) * (-1.0 / D))
            rr = jnp.broadcast_to(r, (L,))
            bag = plsc.load_gather(sid_v, [rr])
            for cc in range(Q // L):
                plsc.addupdate_scatter(
                    acc_v, [bag, col_iota + (cc * L)],
                    x_v[r, pl.ds(cc * L, L)] * u)

        handles = nxt

    pltpu.sync_copy(acc_v, out_hbm.at[wid])


def _sc_segment_weighted_sum(x, x_dists, topology, segment_ids):
    mesh = plsc.VectorSubcoreMesh(core_axis_name="c", subcore_axis_name="s")
    kern = functools.partial(
        pl.kernel,
        out_type=jax.ShapeDtypeStruct((NW, B, Q), _F32),
        mesh=mesh,
        compiler_params=pltpu.CompilerParams(needs_layout_passes=False),
        scratch_types=[
            pltpu.VMEM((CHUNK, Q), _F32),
            pltpu.VMEM((CHUNK, Q), _F32),
            pltpu.VMEM((CHUNK, D), _F32),
            pltpu.VMEM((CHUNK, D), _F32),
            pltpu.VMEM((CHUNK,), jnp.int32),
            pltpu.VMEM((CHUNK,), jnp.int32),
            pltpu.VMEM((D,), _F32),
            pltpu.VMEM((B, Q), _F32),
            pltpu.SemaphoreType.DMA,
            pltpu.SemaphoreType.DMA,
        ],
    )(_sc_body)
    return kern(x, x_dists, topology, segment_ids)


def _tail_body(part_ref, bz_ref, out_ref):
    z16 = jnp.sum(part_ref[...], axis=0)
    bz = bz_ref[...]
    y = jnp.dot(z16, bz, preferred_element_type=_F32)
    b2 = jnp.sqrt(jnp.mean(bz * bz, axis=0, keepdims=True))
    m = jnp.mean(y, axis=0, keepdims=True)
    c = y - m
    var = jnp.sum(c * c, axis=0, keepdims=True) * (1.0 / (B - 1))
    out_ref[...] = b2 * c * lax.rsqrt(var)


def kernel(x, x_dists, topology, beta_z, segment_ids):
    partials = _sc_segment_weighted_sum(
        x, x_dists, topology, segment_ids.astype(jnp.int32))
    bz2 = beta_z.reshape(Q, P * S)
    y = pl.pallas_call(
        _tail_body,
        out_shape=jax.ShapeDtypeStruct((B, P * S), _F32),
    )(partials, bz2)
    return y.reshape(B, P, S)


# speedup vs baseline: 1.2413x; 1.0072x over previous
import functools

import jax
import jax.numpy as jnp
from jax import lax
from jax.experimental import pallas as pl
from jax.experimental.pallas import tpu as pltpu
from jax.experimental.pallas import tpu_sc as plsc

N, Q, P, S, B, D = 32768, 256, 10, 8, 16, 64
NC, NS, L = 2, 16, 16
NW = NC * NS
RPW = N // NW
CHUNK = 128
NCHUNK = RPW // CHUNK

_F32 = jnp.float32

_GDN = lax.GatherDimensionNumbers(
    offset_dims=(), collapsed_slice_dims=(0,), start_index_map=(0,))


def _splat_lane(vec, lane):
    idx = jnp.full((L, 1), lane, jnp.int32)
    return lax.gather(vec, idx, _GDN, slice_sizes=(1,),
                      mode=lax.GatherScatterMode.PROMISE_IN_BOUNDS)


def _hsum_splat(vec):
    x = vec
    for stride in (1, 2, 4, 8):
        idx = (jnp.arange(L, dtype=jnp.int32) ^ stride).reshape(L, 1)
        x = x + lax.gather(x, idx, _GDN, slice_sizes=(1,),
                           mode=lax.GatherScatterMode.PROMISE_IN_BOUNDS)
    return x


def _sort16(v):
    return jnp.sort(v)


def _sort64(d0, d1, d2, d3):
    s0 = _sort16(d0)
    s1d = -_sort16(-d1)
    la, ha = jnp.minimum(s0, s1d), jnp.maximum(s0, s1d)
    a0, a1 = _sort16(la), _sort16(ha)
    s2 = _sort16(d2)
    s3d = -_sort16(-d3)
    lb, hb = jnp.minimum(s2, s3d), jnp.maximum(s2, s3d)
    b0, b1 = -_sort16(-hb), -_sort16(-lb)
    l0, l1 = jnp.minimum(a0, b0), jnp.minimum(a1, b1)
    h0, h1 = jnp.maximum(a0, b0), jnp.maximum(a1, b1)
    o0 = _sort16(jnp.minimum(l0, l1))
    o1 = _sort16(jnp.maximum(l0, l1))
    o2 = _sort16(jnp.minimum(h0, h1))
    o3 = _sort16(jnp.maximum(h0, h1))
    return o0, o1, o2, o3


_LANE_IOTA = tuple(range(L))


def _sc_body(x_hbm, xd_hbm, topo_hbm, seg_hbm, out_hbm,
             x_v0, x_v1, xd_v0, xd_v1, sid_v0, sid_v1,
             topo_v, acc_v, sem0, sem1):
    cid = lax.axis_index("c")
    sid = lax.axis_index("s")
    wid = cid * NS + sid

    pltpu.sync_copy(topo_hbm, topo_v)
    t = [topo_v[pl.ds(k * L, L)] for k in range(4)]
    e = [jnp.exp(tk) for tk in t]
    sden = _hsum_splat(e[0] + e[1] + e[2] + e[3])
    v = [ek / sden for ek in e]
    r0, r1, r2, r3 = _sort64(v[0], v[1], v[2], v[3])

    def zrow(r, carry):
        for cc in range(Q // L):
            acc_v[r, pl.ds(cc * L, L)] = jnp.zeros((L,), _F32)
        return carry
    lax.fori_loop(0, B, zrow, 0)

    base = wid * RPW
    col_iota = jnp.arange(L, dtype=jnp.int32)

    bufs = ((x_v0, xd_v0, sid_v0, sem0), (x_v1, xd_v1, sid_v1, sem1))

    def _start(k):
        xb, db, sb, sem = bufs[k % 2]
        row0 = base + k * CHUNK
        return (pltpu.async_copy(x_hbm.at[pl.ds(row0, CHUNK)], xb, sem),
                pltpu.async_copy(xd_hbm.at[pl.ds(row0, CHUNK)], db, sem),
                pltpu.async_copy(seg_hbm.at[pl.ds(row0, CHUNK)], sb, sem))

    handles = _start(0)
    for k in range(NCHUNK):
        nxt = _start(k + 1) if k + 1 < NCHUNK else None
        for h in handles:
            h.wait()
        x_v, xd_v, sid_v, _ = bufs[k % 2]

        @plsc.parallel_loop(0, CHUNK, 1, unroll=2)
        def row_body(r):
            d0 = xd_v[r, pl.ds(0 * L, L)]
            d1 = xd_v[r, pl.ds(1 * L, L)]
            d2 = xd_v[r, pl.ds(2 * L, L)]
            d3 = xd_v[r, pl.ds(3 * L, L)]
            s0, s1, s2, s3 = _sort64(d0, d1, d2, d3)
            dsum = (jnp.abs(s0 - r0) + jnp.abs(s1 - r1)
                    + jnp.abs(s2 - r2) + jnp.abs(s3 - r3))
            u = jnp.exp(_hsum_splat(dsum) * (-1.0 / D))
            rr = jnp.broadcast_to(r, (L,))
            bag = plsc.load_gather(sid_v, [rr])
            for cc in range(Q // L):
                plsc.addupdate_scatter(
                    acc_v, [bag, col_iota + (cc * L)],
                    x_v[r, pl.ds(cc * L, L)] * u)

        handles = nxt

    pltpu.sync_copy(acc_v, out_hbm.at[wid])


def _sc_segment_weighted_sum(x, x_dists, topology, segment_ids):
    mesh = plsc.VectorSubcoreMesh(core_axis_name="c", subcore_axis_name="s")
    kern = functools.partial(
        pl.kernel,
        out_type=jax.ShapeDtypeStruct((NW, B, Q), _F32),
        mesh=mesh,
        compiler_params=pltpu.CompilerParams(needs_layout_passes=False),
        scratch_types=[
            pltpu.VMEM((CHUNK, Q), _F32),
            pltpu.VMEM((CHUNK, Q), _F32),
            pltpu.VMEM((CHUNK, D), _F32),
            pltpu.VMEM((CHUNK, D), _F32),
            pltpu.VMEM((CHUNK,), jnp.int32),
            pltpu.VMEM((CHUNK,), jnp.int32),
            pltpu.VMEM((D,), _F32),
            pltpu.VMEM((B, Q), _F32),
            pltpu.SemaphoreType.DMA,
            pltpu.SemaphoreType.DMA,
        ],
    )(_sc_body)
    return kern(x, x_dists, topology, segment_ids)


def _tail_body(part_ref, bz_ref, out_ref):
    z16 = jnp.sum(part_ref[...], axis=0)
    bz = bz_ref[...]
    y = jnp.dot(z16, bz, preferred_element_type=_F32)
    b2 = jnp.sqrt(jnp.mean(bz * bz, axis=0, keepdims=True))
    m = jnp.mean(y, axis=0, keepdims=True)
    c = y - m
    var = jnp.sum(c * c, axis=0, keepdims=True) * (1.0 / (B - 1))
    out_ref[...] = b2 * c * lax.rsqrt(var)


def kernel(x, x_dists, topology, beta_z, segment_ids):
    partials = _sc_segment_weighted_sum(
        x, x_dists, topology, segment_ids.astype(jnp.int32))
    bz2 = beta_z.reshape(Q, P * S)
    y = pl.pallas_call(
        _tail_body,
        out_shape=jax.ShapeDtypeStruct((B, P * S), _F32),
    )(partials, bz2)
    return y.reshape(B, P, S)
